# Initial kernel scaffold; baseline (speedup 1.0000x reference)
import jax
import jax.numpy as jnp
from jax.experimental import pallas as pl


def kernel(x, task_id, W1, b1, W2, b2, task_emb, Wg, bg):
    raise NotImplementedError("write your pallas kernel here")


# speedup vs baseline: 1.0717x; 1.0717x over previous
import functools

import jax
import jax.numpy as jnp
from jax import lax
from jax.experimental import pallas as pl
from jax.experimental.pallas import tpu as pltpu
from jax.experimental.pallas import tpu_sc as plsc

BLK = 128
CHUNK = 32


def _gate_body(nb_pad, cap, task_id_ref, x_ref, wg_ref, bg_ref, temb_ref,
               routing_ref, pr0_ref, pr1_ref, pos2_ref, bec_ref, tos_ref,
               lb_ref):
    xm = x_ref[:]
    S, D = xm.shape
    n = S * D
    mean = jnp.sum(xm) / n
    d = xm - mean
    var = jnp.sum(d * d) / n
    xn = d / jnp.sqrt(var + 1e-5)

    wg = wg_ref[:]
    E = wg.shape[0]
    DT = wg.shape[1] - D
    tid = task_id_ref[0]
    trow = temb_ref[pl.ds(tid, 1), :]
    gi = jnp.concatenate([xn, jnp.broadcast_to(trow, (S, DT))], axis=1)
    logits = jnp.dot(gi, wg.T) + bg_ref[:][None, :]

    col = lax.broadcasted_iota(jnp.int32, logits.shape, 1)
    m1 = jnp.max(logits, axis=1)
    i1 = jnp.argmax(logits, axis=1).astype(jnp.int32)
    oh1 = col == i1[:, None]
    neg = jnp.where(oh1, -jnp.inf, logits)
    m2 = jnp.max(neg, axis=1)
    i2 = jnp.argmax(neg, axis=1).astype(jnp.int32)
    oh2 = col == i2[:, None]

    t = jnp.exp(m2 - m1)
    denom = 1.0 + t
    p1 = 1.0 / denom
    p2 = t / denom

    routing_ref[:] = jnp.concatenate([i1[:, None], i2[:, None]], axis=1)
    pr0_ref[:] = jnp.broadcast_to(p1[:, None], (S, 16))
    pr1_ref[:] = jnp.broadcast_to(p2[:, None], (S, 16))

    cnt = oh1.astype(jnp.float32) + oh2.astype(jnp.float32)
    counts = jnp.sum(cnt, axis=0, keepdims=True)

    C = 128
    tr_r = lax.broadcasted_iota(jnp.int32, (C, C), 0)
    tr_c = lax.broadcasted_iota(jnp.int32, (C, C), 1)
    tril_c = (tr_c < tr_r).astype(jnp.float32)
    chunks = []
    csum = jnp.zeros((1, E), jnp.float32)
    for c in range(S // C):
        cc = cnt[c * C:(c + 1) * C]
        chunks.append(jnp.dot(tril_c, cc,
                              preferred_element_type=jnp.float32,
                              precision=lax.Precision.HIGHEST) + csum)
        csum = csum + jnp.sum(cc, axis=0, keepdims=True)
    rank = jnp.concatenate(chunks, axis=0)

    pci = counts.astype(jnp.int32)
    pc = ((pci + (BLK - 1)) // BLK) * BLK
    up_r = lax.broadcasted_iota(jnp.int32, (E, E), 0)
    up_c = lax.broadcasted_iota(jnp.int32, (E, E), 1)
    upper = (up_r < up_c).astype(jnp.float32)
    po = jnp.dot(pc.astype(jnp.float32), upper,
                 preferred_element_type=jnp.float32,
                 precision=lax.Precision.HIGHEST)

    slot_base = po + rank
    pos1 = jnp.sum(jnp.where(oh1, slot_base, 0.0), axis=1)
    pos2 = jnp.sum(jnp.where(oh2, slot_base, 0.0), axis=1)
    pos2_ref[:] = jnp.concatenate(
        [pos1[None, :], pos2[None, :]], axis=0).astype(jnp.int32)

    CW = 512
    tvals = lax.broadcasted_iota(jnp.int32, (S, 1), 0).astype(jnp.float32)
    p0c = pos1[:, None]
    p1c = pos2[:, None]
    toks = []
    for ci in range(cap // CW):
        cols = (lax.broadcasted_iota(jnp.int32, (S, CW), 1)
                .astype(jnp.float32) + float(ci * CW))
        m = jnp.logical_or(p0c == cols, p1c == cols).astype(jnp.float32)
        toks.append(jnp.sum(m * tvals, axis=0, keepdims=True))
    tos_ref[:] = jnp.concatenate(toks, axis=1).astype(jnp.int32)

    bi = lax.broadcasted_iota(jnp.int32, (nb_pad, E), 0) * BLK
    poi = po.astype(jnp.int32)
    ge = (bi >= poi).astype(jnp.int32)
    bec = jnp.sum(ge, axis=1) - 1
    bec_ref[:] = bec[None, :]

    mu = jnp.sum(counts) / E
    usage_mean = mu + 1e-6
    usage_std = jnp.sqrt(jnp.sum((counts - mu) ** 2) / (E - 1))
    lb_ref[:, :] = jnp.broadcast_to((usage_std / usage_mean) ** 2, (1, 1))


def _gate(x2, task_id, Wg, bg, task_emb, nb_pad, cap):
    S, D = x2.shape
    E = Wg.shape[0]
    return pl.pallas_call(
        functools.partial(_gate_body, nb_pad, cap),
        grid=(),
        in_specs=[
            pl.BlockSpec(memory_space=pltpu.SMEM),
            pl.BlockSpec(memory_space=pltpu.VMEM),
            pl.BlockSpec(memory_space=pltpu.VMEM),
            pl.BlockSpec(memory_space=pltpu.VMEM),
            pl.BlockSpec(memory_space=pltpu.VMEM),
        ],
        out_specs=[
            pl.BlockSpec(memory_space=pltpu.VMEM),
            pl.BlockSpec(memory_space=pltpu.VMEM),
            pl.BlockSpec(memory_space=pltpu.VMEM),
            pl.BlockSpec(memory_space=pltpu.VMEM),
            pl.BlockSpec(memory_space=pltpu.VMEM),
            pl.BlockSpec(memory_space=pltpu.VMEM),
            pl.BlockSpec(memory_space=pltpu.VMEM),
        ],
        out_shape=[
            jax.ShapeDtypeStruct((S, 2), jnp.int32),
            jax.ShapeDtypeStruct((S, 16), jnp.float32),
            jax.ShapeDtypeStruct((S, 16), jnp.float32),
            jax.ShapeDtypeStruct((2, S), jnp.int32),
            jax.ShapeDtypeStruct((1, nb_pad), jnp.int32),
            jax.ShapeDtypeStruct((1, cap), jnp.int32),
            jax.ShapeDtypeStruct((1, 1), jnp.float32),
        ],
    )(task_id, x2, Wg, bg, task_emb)


def _dispatch(x2, tos, cap):
    S, D = x2.shape
    info = plsc.get_sparse_core_info()
    nw = info.num_cores * info.num_subcores
    n_chunks = cap // CHUNK
    per_w = -(-n_chunks // nw)
    mesh = plsc.VectorSubcoreMesh(core_axis_name="c", subcore_axis_name="s")

    @functools.partial(
        pl.kernel, mesh=mesh,
        out_type=jax.ShapeDtypeStruct((cap, D), jnp.float32),
        scratch_types=[pltpu.VMEM((CHUNK, D), jnp.float32),
                       pltpu.VMEM((CHUNK,), jnp.int32)],
    )
    def disp(x_hbm, tos_hbm, xg_hbm, xbuf, idx):
        wid = lax.axis_index("s") * info.num_cores + lax.axis_index("c")
        for c in range(per_w):
            cid = wid + c * nw
            @pl.when(cid < n_chunks)
            def _():
                base = cid * CHUNK
                pltpu.sync_copy(tos_hbm.at[pl.ds(base, CHUNK)], idx)
                pltpu.sync_copy(x_hbm.at[idx], xbuf)
                pltpu.sync_copy(xbuf, xg_hbm.at[pl.ds(base, CHUNK)])

    return disp(x2, tos)


def _ffn1_body(bec_ref, xg_ref, w1_ref, b1_ref, h_ref, wcache, ecache):
    b = pl.program_id(0)
    e = bec_ref[b]

    @pl.when(jnp.logical_or(b == 0, e != ecache[0]))
    def _():
        wcache[...] = w1_ref[0].astype(jnp.bfloat16)
        ecache[0] = e

    xb = xg_ref[...].astype(jnp.bfloat16)
    h = lax.dot_general(xb, wcache[...], (((1,), (1,)), ((), ())),
                        preferred_element_type=jnp.float32)
    h = h + b1_ref[0]
    h_ref[...] = (h * jax.nn.sigmoid(h)).astype(jnp.bfloat16)


def _ffn1(bec, xg, W1, b1r):
    cap, D = xg.shape
    E, FF, _ = W1.shape
    nb = cap // BLK
    return pl.pallas_call(
        _ffn1_body,
        grid_spec=pltpu.PrefetchScalarGridSpec(
            num_scalar_prefetch=1,
            grid=(nb,),
            in_specs=[
                pl.BlockSpec((BLK, D), lambda b, bec: (b, 0)),
                pl.BlockSpec((1, FF, D), lambda b, bec: (bec[b], 0, 0)),
                pl.BlockSpec((1, 1, FF), lambda b, bec: (bec[b], 0, 0)),
            ],
            out_specs=pl.BlockSpec((BLK, FF), lambda b, bec: (b, 0)),
            scratch_shapes=[pltpu.VMEM((FF, D), jnp.bfloat16),
                            pltpu.SMEM((1,), jnp.int32)],
        ),
        out_shape=jax.ShapeDtypeStruct((cap, FF), jnp.bfloat16),
    )(bec, xg, W1, b1r)


def _ffn2_body(bec_ref, h_ref, w2_ref, b2_ref, yw_ref, wcache, ecache):
    b = pl.program_id(0)
    e = bec_ref[b]

    @pl.when(jnp.logical_or(b == 0, e != ecache[0]))
    def _():
        wcache[...] = w2_ref[0].astype(jnp.bfloat16)
        ecache[0] = e

    y = lax.dot_general(h_ref[...], wcache[...], (((1,), (1,)), ((), ())),
                        preferred_element_type=jnp.float32)
    yw_ref[...] = y + b2_ref[0]


def _ffn2(bec, h, W2, b2r):
    cap, FF = h.shape
    E, D, _ = W2.shape
    nb = cap // BLK
    return pl.pallas_call(
        _ffn2_body,
        grid_spec=pltpu.PrefetchScalarGridSpec(
            num_scalar_prefetch=1,
            grid=(nb,),
            in_specs=[
                pl.BlockSpec((BLK, FF), lambda b, bec: (b, 0)),
                pl.BlockSpec((1, D, FF), lambda b, bec: (bec[b], 0, 0)),
                pl.BlockSpec((1, 1, D), lambda b, bec: (bec[b], 0, 0)),
            ],
            out_specs=pl.BlockSpec((BLK, D), lambda b, bec: (b, 0)),
            scratch_shapes=[pltpu.VMEM((D, FF), jnp.bfloat16),
                            pltpu.SMEM((1,), jnp.int32)],
        ),
        out_shape=jax.ShapeDtypeStruct((cap, D), jnp.float32),
    )(bec, h, W2, b2r)


def _combine(yw, pos0, pos1, pr0, pr1, S):
    cap, D = yw.shape
    info = plsc.get_sparse_core_info()
    nw = info.num_cores * info.num_subcores
    n_chunks = S // CHUNK
    per_w = -(-n_chunks // nw)
    mesh = plsc.VectorSubcoreMesh(core_axis_name="c", subcore_axis_name="s")

    @functools.partial(
        pl.kernel, mesh=mesh,
        out_type=jax.ShapeDtypeStruct((S, D), jnp.float32),
        scratch_types=[pltpu.VMEM((CHUNK, D), jnp.float32),
                       pltpu.VMEM((CHUNK, D), jnp.float32),
                       pltpu.VMEM((CHUNK,), jnp.int32),
                       pltpu.VMEM((CHUNK,), jnp.int32),
                       pltpu.VMEM((CHUNK, 16), jnp.float32),
                       pltpu.VMEM((CHUNK, 16), jnp.float32)],
    )
    def comb(yw_hbm, p0_hbm, p1_hbm, q0_hbm, q1_hbm, out_hbm,
             buf0, buf1, i0, i1, q0, q1):
        wid = lax.axis_index("s") * info.num_cores + lax.axis_index("c")
        ncol = D // 16
        for c in range(per_w):
            cid = wid + c * nw
            @pl.when(cid < n_chunks)
            def _():
                base = cid * CHUNK
                pltpu.sync_copy(p0_hbm.at[pl.ds(base, CHUNK)], i0)
                pltpu.sync_copy(p1_hbm.at[pl.ds(base, CHUNK)], i1)
                pltpu.sync_copy(q0_hbm.at[pl.ds(base, CHUNK)], q0)
                pltpu.sync_copy(q1_hbm.at[pl.ds(base, CHUNK)], q1)
                pltpu.sync_copy(yw_hbm.at[i0], buf0)
                pltpu.sync_copy(yw_hbm.at[i1], buf1)

                def row_add(r, _):
                    s0 = q0[r, :]
                    s1 = q1[r, :]
                    for u in range(ncol):
                        sl = pl.ds(u * 16, 16)
                        buf0[r, sl] = buf0[r, sl] * s0 + buf1[r, sl] * s1
                    return 0

                lax.fori_loop(0, CHUNK, row_add, 0)
                pltpu.sync_copy(buf0, out_hbm.at[pl.ds(base, CHUNK)])

    return comb(yw, pos0, pos1, pr0, pr1)


def kernel(x, task_id, W1, b1, W2, b2, task_emb, Wg, bg):
    B, S, D = x.shape
    E, FF, _ = W1.shape
    nb = (2 * S + E * (BLK - 1) + BLK - 1) // BLK
    cap = nb * BLK
    nb_pad = -(-nb // 128) * 128

    x2 = x.reshape(S, D)
    routing, pr0, pr1, pos2, bec2, tos2, lb = _gate(
        x2, task_id, Wg, bg, task_emb, nb_pad, cap)

    pos0 = pos2[0]
    pos1 = pos2[1]
    bec = bec2[0, :nb]
    tos = tos2[0]

    xg = _dispatch(x2, tos, cap)
    h = _ffn1(bec, xg, W1, b1.reshape(E, 1, FF))
    yw = _ffn2(bec, h, W2, b2.reshape(E, 1, D))
    out = _combine(yw, pos0, pos1, pr0, pr1, S)

    final_output = out.reshape(B, S, D)
    lb_loss = lb.reshape(())
    routing_indices = routing.reshape(B, S, 2)
    return final_output, lb_loss, routing_indices, task_id
